# Initial kernel scaffold; baseline (speedup 1.0000x reference)
import jax
import jax.numpy as jnp
from jax.experimental import pallas as pl


def kernel(preds, targets, grid_scale):
    raise NotImplementedError("write your pallas kernel here")


# speedup vs baseline: 363.1973x; 363.1973x over previous
import functools
import math

import jax
import jax.numpy as jnp
from jax import lax
from jax.experimental import pallas as pl
from jax.experimental.pallas import tpu as pltpu
from jax.experimental.pallas import tpu_sc as plsc

NUM_CLASSES_ = 19
GS = 16
NC = 2
NS = 16
NW = NC * NS
LANES = 16

ROWS_TOTAL = 16 * 1024
ROW_W = 1024
ROWS_PER_W = ROWS_TOTAL // NW
CH = 32
NCH = ROWS_PER_W // CH
NPAIR = NCH // 2


def _row_or(buf, slot, r, a):
    zero = jnp.zeros((LANES,), jnp.int32)

    def body(x8, carry):
        a0, a1, a2, a3 = carry
        base = x8 * 128
        acc = [a0, a1, a2, a3]
        for k in range(8):
            v = buf[slot, r, pl.ds(base + k * LANES, LANES)]
            acc[k % 4] = acc[k % 4] | (1 << v)
        return tuple(acc)

    a0, a1, a2, a3 = lax.fori_loop(0, ROW_W // (8 * LANES), body,
                                   (a, zero, zero, zero))
    return (a0 | a1) | (a2 | a3)


def _process_slot(buf, acc_ref, slot):
    def row_body(r, carry):
        i = lax.rem(r, GS)
        acc_ref[i] = _row_or(buf, slot, r, acc_ref[i])
        return carry

    lax.fori_loop(0, CH, row_body, 0)


def _sc_body(t_hbm, out_hbm, buf, acc_ref, sem0, sem1):
    cid = lax.axis_index("c")
    sid = lax.axis_index("s")
    wid = sid * NC + cid
    base = wid * ROWS_PER_W
    b = wid // 2
    half = wid % 2

    def copy_in(chunk_rows, slot, sem):
        return pltpu.make_async_copy(
            t_hbm.at[pl.ds(base + chunk_rows, CH)], buf.at[slot], sem)

    copy_in(0, 0, sem0).start()
    copy_in(CH, 1, sem1).start()

    for i in range(GS):
        acc_ref[i] = jnp.zeros((LANES,), jnp.int32)

    def pair_body(p, carry):
        copy_in(0, 0, sem0).wait()
        _process_slot(buf, acc_ref, 0)

        @pl.when(p < NPAIR - 1)
        def _():
            copy_in((2 * p + 2) * CH, 0, sem0).start()

        copy_in(0, 1, sem1).wait()
        _process_slot(buf, acc_ref, 1)

        @pl.when(p < NPAIR - 1)
        def _():
            copy_in((2 * p + 3) * CH, 1, sem1).start()

        return carry

    lax.fori_loop(0, NPAIR, pair_body, 0)
    pltpu.sync_copy(acc_ref, out_hbm.at[half, b])


_sc_presence = functools.partial(
    pl.kernel,
    mesh=plsc.VectorSubcoreMesh(core_axis_name="c", subcore_axis_name="s"),
    out_type=jax.ShapeDtypeStruct((2, 16, GS, GS), jnp.int32),
    scratch_types=[
        pltpu.VMEM((2, CH, ROW_W), jnp.int32),
        pltpu.VMEM((GS, GS), jnp.int32),
        pltpu.SemaphoreType.DMA,
        pltpu.SemaphoreType.DMA,
    ],
)(_sc_body)


def _bce_body(off_ref, predsT_ref, masks_ref, out_ref):
    off = off_ref[0]
    m = masks_ref[0:1, :] | masks_ref[1:2, :]
    m_pos = m << jnp.maximum(off, 0)
    m_neg = lax.shift_right_logical(m, jnp.maximum(-off, 0))
    m = jnp.where(off >= 0, m_pos, m_neg)

    n, rows = predsT_ref.shape
    c = lax.broadcasted_iota(jnp.int32, (n, rows), 0)
    t = ((m >> c) & 1).astype(jnp.float32)
    x = predsT_ref[...]
    p = jax.nn.sigmoid(x)
    log_p = jnp.maximum(jnp.log(p), -100.0)
    log_1mp = jnp.maximum(jnp.log(1.0 - p), -100.0)
    loss = t * log_p + (1.0 - t) * log_1mp
    out_ref[0, 0] = -jnp.sum(loss) * (1.0 / float(n * rows))


def kernel(preds, targets, grid_scale):
    gs = int(math.isqrt(preds.shape[0] // targets.shape[0]))
    off = (jnp.asarray(grid_scale, jnp.int32) - gs).reshape((1,))

    t2 = targets.reshape(ROWS_TOTAL, ROW_W)
    partial = _sc_presence(t2)
    masks = partial.reshape(2, preds.shape[0])

    loss = pl.pallas_call(
        _bce_body,
        out_shape=jax.ShapeDtypeStruct((1, 1), jnp.float32),
        in_specs=[
            pl.BlockSpec(memory_space=pltpu.SMEM),
            pl.BlockSpec(memory_space=pltpu.VMEM),
            pl.BlockSpec(memory_space=pltpu.VMEM),
        ],
        out_specs=pl.BlockSpec(memory_space=pltpu.SMEM),
    )(off, preds.T, masks)
    return loss[0, 0]
